# baseline (speedup 1.0000x reference)
import functools

import jax
import jax.numpy as jnp
from jax import lax
from jax.experimental import pallas as pl
from jax.experimental.pallas import tpu as pltpu
from jax.experimental.pallas import tpu_sc as plsc

VOCAB = 1000000
D = 32
CHUNK = 2048
GROUP = 128
LANES = 16


@functools.partial(jax.jit, static_argnames=("n",))
def _lookup(table, idx, n):
    info = plsc.get_sparse_core_info()
    nc, ns = info.num_cores, info.num_subcores
    nw = nc * ns
    per_w = n // nw
    n_chunks = per_w // CHUNK
    mesh = plsc.VectorSubcoreMesh(core_axis_name="c", subcore_axis_name="s")

    @functools.partial(
        pl.kernel,
        mesh=mesh,
        compiler_params=pltpu.CompilerParams(use_tc_tiling_on_sc=False),
        out_type=jax.ShapeDtypeStruct((n, D), jnp.float32),
        scratch_types=[
            pltpu.VMEM((CHUNK,), jnp.int32),
            pltpu.VMEM((CHUNK,), jnp.int32),
            pltpu.VMEM((CHUNK,), jnp.float32),
            pltpu.VMEM((CHUNK, D), jnp.float32),
            pltpu.SemaphoreType.DMA,
        ],
    )
    def k(table_hbm, idx_hbm, out_hbm, idx_raw, idx_safe, maskf, rows, sem):
        wid = lax.axis_index("s") * nc + lax.axis_index("c")
        base0 = wid * per_w

        def chunk_body(i, carry):
            base = base0 + i * CHUNK
            pltpu.sync_copy(idx_hbm.at[pl.ds(base, CHUNK)], idx_raw)

            def fix(v, c):
                iv = idx_raw[pl.ds(v * LANES, LANES)]
                idx_safe[pl.ds(v * LANES, LANES)] = jnp.maximum(iv - VOCAB, 0)
                maskf[pl.ds(v * LANES, LANES)] = jnp.where(
                    iv >= VOCAB, 1.0, 0.0
                ).astype(jnp.float32)
                return c

            lax.fori_loop(0, CHUNK // LANES, fix, 0)

            copies = [
                pltpu.async_copy(
                    table_hbm.at[idx_safe.at[pl.ds(g * GROUP, GROUP)]],
                    rows.at[pl.ds(g * GROUP, GROUP)],
                    sem,
                )
                for g in range(CHUNK // GROUP)
            ]
            for cp in copies:
                cp.wait()

            def zero(v, c):
                mvec = maskf[pl.ds(v * LANES, LANES)]
                for j in range(LANES):
                    r = v * LANES + j
                    m = jnp.full((LANES,), mvec[j], jnp.float32)
                    for h in range(D // LANES):
                        sl = pl.ds(h * LANES, LANES)
                        rows[r, sl] = rows[r, sl] * m
                return c

            lax.fori_loop(0, CHUNK // LANES, zero, 0)

            pltpu.sync_copy(rows, out_hbm.at[pl.ds(base, CHUNK)])
            return carry

        lax.fori_loop(0, n_chunks, chunk_body, 0)

    return k(table, idx)


def kernel(inputs, embeddings):
    b, s = inputs.shape
    n = b * s
    idx = inputs.reshape(n).astype(jnp.int32)
    out = _lookup(embeddings, idx, n)
    return out.reshape(b, s, D)


# speedup vs baseline: 6.7003x; 6.7003x over previous
import functools

import jax
import jax.numpy as jnp
from jax import lax
from jax.experimental import pallas as pl
from jax.experimental.pallas import tpu as pltpu
from jax.experimental.pallas import tpu_sc as plsc

VOCAB = 1000000
D = 32
CHUNK = 1024
GROUP = 128
LANES = 16
NBUF = 2
SENTINEL = -1


@functools.partial(jax.jit, static_argnames=("n",))
def _lookup(table, idx, n):
    info = plsc.get_sparse_core_info()
    nc, ns = info.num_cores, info.num_subcores
    nw = nc * ns
    per_w = n // nw
    n_chunks = per_w // CHUNK
    mesh = plsc.VectorSubcoreMesh(core_axis_name="c", subcore_axis_name="s")

    @functools.partial(
        pl.kernel,
        mesh=mesh,
        compiler_params=pltpu.CompilerParams(use_tc_tiling_on_sc=False),
        out_type=jax.ShapeDtypeStruct((n, D), jnp.float32),
        scratch_types=[
            pltpu.VMEM((NBUF, CHUNK), jnp.int32),
            pltpu.VMEM((NBUF, CHUNK, D), jnp.float32),
            pltpu.SemaphoreType.DMA((NBUF,)),
            pltpu.SemaphoreType.DMA((NBUF,)),
            pltpu.SemaphoreType.DMA((NBUF,)),
        ],
    )
    def k(table_hbm, idx_hbm, out_hbm, idx_b, rows_b, sem_i, sem_g, sem_o):
        wid = lax.axis_index("s") * nc + lax.axis_index("c")
        base0 = wid * per_w
        zeros16 = jnp.zeros((LANES,), jnp.float32)

        def start_idx(i, s):
            pltpu.async_copy(
                idx_hbm.at[pl.ds(base0 + i * CHUNK, CHUNK)],
                idx_b.at[s],
                sem_i.at[s],
            )

        for i in range(NBUF):
            start_idx(i, i)

        def chunk_body(i, carry):
            s = lax.rem(i, NBUF)
            base = base0 + i * CHUNK
            idx_s = idx_b.at[s]
            rows_s = rows_b.at[s]

            pltpu.make_async_copy(
                idx_hbm.at[pl.ds(base, CHUNK)], idx_s, sem_i.at[s]
            ).wait()

            def fix(v, c):
                sl = pl.ds(v * LANES, LANES)
                iv = idx_s[sl]
                idx_s[sl] = jnp.where(iv >= VOCAB, iv - VOCAB, SENTINEL)
                return c

            lax.fori_loop(0, CHUNK // LANES, fix, 0)

            @pl.when(i >= NBUF)
            def _():
                pltpu.make_async_copy(
                    rows_s,
                    out_hbm.at[pl.ds(base0 + (i - NBUF) * CHUNK, CHUNK)],
                    sem_o.at[s],
                ).wait()

            def clear(v, c):
                rows_s[v, pl.ds(0, LANES)] = zeros16
                rows_s[v, pl.ds(LANES, LANES)] = zeros16
                return c

            lax.fori_loop(0, CHUNK, clear, 0)

            copies = [
                pltpu.async_copy(
                    table_hbm.at[
                        plsc.Indices(
                            idx_s.at[pl.ds(g * GROUP, GROUP)],
                            ignored_value=SENTINEL,
                        )
                    ],
                    rows_s.at[pl.ds(g * GROUP, GROUP)],
                    sem_g.at[s],
                )
                for g in range(CHUNK // GROUP)
            ]
            for cp in copies:
                cp.wait()

            pltpu.async_copy(
                rows_s, out_hbm.at[pl.ds(base, CHUNK)], sem_o.at[s]
            )

            @pl.when(i + NBUF < n_chunks)
            def _():
                start_idx(i + NBUF, s)

            return carry

        lax.fori_loop(0, n_chunks, chunk_body, 0)

        for j in range(NBUF):
            i = n_chunks - NBUF + j
            s = i % NBUF
            pltpu.make_async_copy(
                rows_b.at[s],
                out_hbm.at[pl.ds(base0 + i * CHUNK, CHUNK)],
                sem_o.at[s],
            ).wait()

    return k(table, idx)


def kernel(inputs, embeddings):
    b, s = inputs.shape
    n = b * s
    idx = inputs.reshape(n).astype(jnp.int32)
    out = _lookup(embeddings, idx, n)
    return out.reshape(b, s, D)


# speedup vs baseline: 6.7498x; 1.0074x over previous
import functools

import jax
import jax.numpy as jnp
from jax import lax
from jax.experimental import pallas as pl
from jax.experimental.pallas import tpu as pltpu
from jax.experimental.pallas import tpu_sc as plsc

VOCAB = 1000000
D = 32
ROWS_PER_CHUNK = 8
LANES = 16
NBUF = 2
SENTINEL = -1


@functools.partial(jax.jit, static_argnames=("b", "s"))
def _lookup(table, idx, b, s):
    n = b * s
    chunk = ROWS_PER_CHUNK * s
    info = plsc.get_sparse_core_info()
    nc, ns = info.num_cores, info.num_subcores
    nw = nc * ns
    rows_w = b // nw
    per_w = rows_w * s
    n_chunks = rows_w // ROWS_PER_CHUNK
    bounds = list(range(0, chunk, 128)) + [chunk]
    groups = [(lo, hi - lo) for lo, hi in zip(bounds[:-1], bounds[1:])]
    mesh = plsc.VectorSubcoreMesh(core_axis_name="c", subcore_axis_name="s")

    @functools.partial(
        pl.kernel,
        mesh=mesh,
        compiler_params=pltpu.CompilerParams(use_tc_tiling_on_sc=False),
        out_type=jax.ShapeDtypeStruct((b, s, D), jnp.float32),
        scratch_types=[
            pltpu.VMEM((NBUF, chunk), jnp.int32),
            pltpu.VMEM((NBUF, chunk, D), jnp.float32),
            pltpu.SemaphoreType.DMA((NBUF,)),
            pltpu.SemaphoreType.DMA((NBUF,)),
            pltpu.SemaphoreType.DMA((NBUF,)),
        ],
    )
    def k(table_hbm, idx_hbm, out_hbm, idx_b, rows_b, sem_i, sem_g, sem_o):
        wid = lax.axis_index("s") * nc + lax.axis_index("c")
        base0 = wid * per_w
        row0 = wid * rows_w
        zeros16 = jnp.zeros((LANES,), jnp.float32)

        def start_idx(i, sl):
            pltpu.async_copy(
                idx_hbm.at[pl.ds(base0 + i * chunk, chunk)],
                idx_b.at[sl],
                sem_i.at[sl],
            )

        def out_copies(i, sl, start):
            rb = row0 + i * ROWS_PER_CHUNK
            for r in range(ROWS_PER_CHUNK):
                cp = pltpu.make_async_copy(
                    rows_b.at[sl].at[pl.ds(r * s, s)],
                    out_hbm.at[rb + r],
                    sem_o.at[sl],
                )
                if start:
                    cp.start()
                else:
                    cp.wait()

        for i in range(NBUF):
            start_idx(i, i)

        def chunk_body(i, carry):
            sl = lax.rem(i, NBUF)
            idx_s = idx_b.at[sl]
            rows_s = rows_b.at[sl]

            pltpu.make_async_copy(
                idx_hbm.at[pl.ds(base0 + i * chunk, chunk)], idx_s, sem_i.at[sl]
            ).wait()

            def fix(v, c):
                vs = pl.ds(v * LANES, LANES)
                iv = idx_s[vs]
                idx_s[vs] = jnp.where(iv >= VOCAB, iv - VOCAB, SENTINEL)
                return c

            lax.fori_loop(0, chunk // LANES, fix, 0)

            @pl.when(i >= NBUF)
            def _():
                out_copies(i - NBUF, sl, start=False)

            def clear(v, c):
                rows_s[v, pl.ds(0, LANES)] = zeros16
                rows_s[v, pl.ds(LANES, LANES)] = zeros16
                return c

            lax.fori_loop(0, chunk, clear, 0)

            copies = [
                pltpu.async_copy(
                    table_hbm.at[
                        plsc.Indices(
                            idx_s.at[pl.ds(lo, sz)], ignored_value=SENTINEL
                        )
                    ],
                    rows_s.at[pl.ds(lo, sz)],
                    sem_g.at[sl],
                )
                for lo, sz in groups
            ]
            for cp in copies:
                cp.wait()

            out_copies(i, sl, start=True)

            @pl.when(i + NBUF < n_chunks)
            def _():
                start_idx(i + NBUF, sl)

            return carry

        lax.fori_loop(0, n_chunks, chunk_body, 0)

        for j in range(NBUF):
            i = n_chunks - NBUF + j
            out_copies(i, i % NBUF, start=False)

    return k(table, idx)


def kernel(inputs, embeddings):
    b, s = inputs.shape
    idx = inputs.reshape(b * s).astype(jnp.int32)
    return _lookup(embeddings, idx, b, s)


# speedup vs baseline: 6.7557x; 1.0009x over previous
import functools

import jax
import jax.numpy as jnp
from jax import lax
from jax.experimental import pallas as pl
from jax.experimental.pallas import tpu as pltpu
from jax.experimental.pallas import tpu_sc as plsc

VOCAB = 1000000
D = 32
RPC = 8
LANES = 16
NBUF = 2
SENTINEL = -1


@functools.partial(jax.jit, static_argnames=("b", "s"))
def _lookup(table, idx2, b, s):
    sp = -(-s // LANES) * LANES
    info = plsc.get_sparse_core_info()
    nc, ns = info.num_cores, info.num_subcores
    nw = nc * ns
    rows_w = b // nw
    n_chunks = rows_w // RPC
    bounds = list(range(0, s, 128)) + [s]
    groups = [(lo, hi - lo) for lo, hi in zip(bounds[:-1], bounds[1:])]
    voffs = [o * LANES for o in range(s // LANES)]
    if s % LANES:
        voffs.append(s - LANES)
    mesh = plsc.VectorSubcoreMesh(core_axis_name="c", subcore_axis_name="s")

    @functools.partial(
        pl.kernel,
        mesh=mesh,
        compiler_params=pltpu.CompilerParams(use_tc_tiling_on_sc=False),
        out_type=jax.ShapeDtypeStruct((b, s, D), jnp.float32),
        scratch_types=[
            pltpu.VMEM((NBUF, RPC, s), jnp.int32),
            pltpu.VMEM((NBUF, RPC, sp), jnp.int32),
            pltpu.VMEM((NBUF, RPC, s, D), jnp.float32),
            pltpu.SemaphoreType.DMA((NBUF,)),
            pltpu.SemaphoreType.DMA((NBUF,)),
            pltpu.SemaphoreType.DMA((NBUF,)),
        ],
    )
    def k(table_hbm, idx_hbm, out_hbm, raw_b, code_b, rows_b, sem_i, sem_g,
          sem_o):
        wid = lax.axis_index("s") * nc + lax.axis_index("c")
        row0 = wid * rows_w
        zeros16 = jnp.zeros((LANES,), jnp.float32)

        def start_idx(i, sl):
            pltpu.async_copy(
                idx_hbm.at[pl.ds(row0 + i * RPC, RPC)],
                raw_b.at[sl],
                sem_i.at[sl],
            )

        def out_copies(i, sl, start):
            rb = row0 + i * RPC
            for r in range(RPC):
                cp = pltpu.make_async_copy(
                    rows_b.at[sl].at[r],
                    out_hbm.at[rb + r],
                    sem_o.at[sl],
                )
                if start:
                    cp.start()
                else:
                    cp.wait()

        for i in range(NBUF):
            start_idx(i, i)

        def chunk_body(i, carry):
            sl = lax.rem(i, NBUF)
            raw_s = raw_b.at[sl]
            code_s = code_b.at[sl]
            rows_s = rows_b.at[sl]

            pltpu.make_async_copy(
                idx_hbm.at[pl.ds(row0 + i * RPC, RPC)], raw_s, sem_i.at[sl]
            ).wait()

            for r in range(RPC):
                for o in voffs:
                    iv = raw_s[r, pl.ds(o, LANES)]
                    code_s[r, pl.ds(o, LANES)] = jnp.where(
                        iv >= VOCAB, iv - VOCAB, SENTINEL
                    )

            @pl.when(i >= NBUF)
            def _():
                out_copies(i - NBUF, sl, start=False)

            def clear_row(r):
                def body(v, c):
                    rows_s[r, v, pl.ds(0, LANES)] = zeros16
                    rows_s[r, v, pl.ds(LANES, LANES)] = zeros16
                    return c

                lax.fori_loop(0, s, body, 0)

            for r in range(RPC):
                clear_row(r)

            copies = [
                pltpu.async_copy(
                    table_hbm.at[
                        plsc.Indices(
                            code_s.at[r].at[pl.ds(lo, sz)],
                            ignored_value=SENTINEL,
                        )
                    ],
                    rows_s.at[r].at[pl.ds(lo, sz)],
                    sem_g.at[sl],
                )
                for r in range(RPC)
                for lo, sz in groups
            ]
            for cp in copies:
                cp.wait()

            out_copies(i, sl, start=True)

            @pl.when(i + NBUF < n_chunks)
            def _():
                start_idx(i + NBUF, sl)

            return carry

        lax.fori_loop(0, n_chunks, chunk_body, 0)

        for j in range(NBUF):
            i = n_chunks - NBUF + j
            out_copies(i, i % NBUF, start=False)

    return k(table, idx2)


def kernel(inputs, embeddings):
    b, s = inputs.shape
    return _lookup(embeddings, inputs.astype(jnp.int32), b, s)


# speedup vs baseline: 6.9318x; 1.0261x over previous
import functools

import jax
import jax.numpy as jnp
from jax import lax
from jax.experimental import pallas as pl
from jax.experimental.pallas import tpu as pltpu
from jax.experimental.pallas import tpu_sc as plsc

VOCAB = 1000000
D = 32
RPC = 8
LANES = 16
NBUF = 2
SENTINEL = -1


@functools.partial(jax.jit, static_argnames=("b", "s"))
def _lookup(table, idx2, b, s):
    sp = -(-s // LANES) * LANES
    info = plsc.get_sparse_core_info()
    nc, ns = info.num_cores, info.num_subcores
    nw = nc * ns
    rows_w = b // nw
    n_chunks = rows_w // RPC
    bounds = list(range(0, s, 128)) + [s]
    groups = [(lo, hi - lo) for lo, hi in zip(bounds[:-1], bounds[1:])]
    voffs = [o * LANES for o in range(s // LANES)]
    if s % LANES:
        voffs.append(s - LANES)
    mesh = plsc.VectorSubcoreMesh(core_axis_name="c", subcore_axis_name="s")

    @functools.partial(
        pl.kernel,
        mesh=mesh,
        compiler_params=pltpu.CompilerParams(use_tc_tiling_on_sc=False),
        out_type=jax.ShapeDtypeStruct((b, s, D), jnp.float32),
        scratch_types=[
            pltpu.VMEM((NBUF, RPC, s), jnp.int32),
            pltpu.VMEM((NBUF, RPC, sp), jnp.int32),
            pltpu.VMEM((NBUF, RPC, s, D), jnp.float32),
            pltpu.SemaphoreType.DMA((NBUF,)),
            pltpu.SemaphoreType.DMA((NBUF,)),
            pltpu.SemaphoreType.DMA((NBUF,)),
        ],
    )
    def k(table_hbm, idx_hbm, out_hbm, raw_b, code_b, rows_b, sem_i, sem_g,
          sem_o):
        wid = lax.axis_index("s") * nc + lax.axis_index("c")
        row0 = wid * rows_w
        zeros16 = jnp.zeros((LANES,), jnp.float32)

        def start_idx(i, sl):
            pltpu.async_copy(
                idx_hbm.at[pl.ds(row0 + i * RPC, RPC)],
                raw_b.at[sl],
                sem_i.at[sl],
            )

        def out_copies(i, sl, start):
            rb = row0 + i * RPC
            for r in range(RPC):
                cp = pltpu.make_async_copy(
                    rows_b.at[sl].at[r],
                    out_hbm.at[rb + r],
                    sem_o.at[sl],
                )
                if start:
                    cp.start()
                else:
                    cp.wait()

        def gathers(i, sl, start):
            code_s = code_b.at[sl]
            rows_s = rows_b.at[sl]
            for r in range(RPC):
                for lo, sz in groups:
                    cp = pltpu.make_async_copy(
                        table_hbm.at[
                            plsc.Indices(
                                code_s.at[r].at[pl.ds(lo, sz)],
                                ignored_value=SENTINEL,
                            )
                        ],
                        rows_s.at[r].at[pl.ds(lo, sz)],
                        sem_g.at[sl],
                    )
                    if start:
                        cp.start()
                    else:
                        cp.wait()

        def prep(i):
            sl = lax.rem(i, NBUF)
            raw_s = raw_b.at[sl]
            code_s = code_b.at[sl]
            rows_s = rows_b.at[sl]

            pltpu.make_async_copy(
                idx_hbm.at[pl.ds(row0 + i * RPC, RPC)], raw_s, sem_i.at[sl]
            ).wait()

            for r in range(RPC):
                for o in voffs:
                    iv = raw_s[r, pl.ds(o, LANES)]
                    code_s[r, pl.ds(o, LANES)] = jnp.where(
                        iv >= VOCAB, iv - VOCAB, SENTINEL
                    )

            @pl.when(i >= NBUF)
            def _():
                out_copies(i - NBUF, sl, start=False)

            def clear_row(r):
                def body(v, c):
                    rows_s[r, v, pl.ds(0, LANES)] = zeros16
                    rows_s[r, v, pl.ds(LANES, LANES)] = zeros16
                    return c

                lax.fori_loop(0, s, body, 0)

            for r in range(RPC):
                clear_row(r)

            gathers(i, sl, start=True)

            @pl.when(i + NBUF < n_chunks)
            def _():
                start_idx(i + NBUF, sl)

        for i in range(NBUF):
            start_idx(i, i)
        prep(0)

        def chunk_body(i, carry):
            @pl.when(i + 1 < n_chunks)
            def _():
                prep(i + 1)

            gathers(i, lax.rem(i, NBUF), start=False)
            out_copies(i, lax.rem(i, NBUF), start=True)
            return carry

        lax.fori_loop(0, n_chunks, chunk_body, 0)

        for j in range(NBUF):
            i = n_chunks - NBUF + j
            out_copies(i, i % NBUF, start=False)

    return k(table, idx2)


def kernel(inputs, embeddings):
    b, s = inputs.shape
    return _lookup(embeddings, inputs.astype(jnp.int32), b, s)


# speedup vs baseline: 7.7647x; 1.1202x over previous
import functools

import jax
import jax.numpy as jnp
from jax import lax
from jax.experimental import pallas as pl
from jax.experimental.pallas import tpu as pltpu
from jax.experimental.pallas import tpu_sc as plsc

VOCAB = 1000000
D = 32
RPC = 8
LANES = 16
NBUF = 2
SENTINEL = -1


@functools.partial(jax.jit, static_argnames=("b", "s"))
def _lookup(table, idx2, b, s):
    sp = -(-s // LANES) * LANES
    info = plsc.get_sparse_core_info()
    nc, ns = info.num_cores, info.num_subcores
    nw = nc * ns
    rows_w = b // nw
    n_chunks = rows_w // RPC
    bounds = list(range(0, s, 128)) + [s]
    groups = [(lo, hi - lo) for lo, hi in zip(bounds[:-1], bounds[1:])]
    voffs = [o * LANES for o in range(s // LANES)]
    if s % LANES:
        voffs.append(s - LANES)
    mesh = plsc.VectorSubcoreMesh(core_axis_name="c", subcore_axis_name="s")

    @functools.partial(
        pl.kernel,
        mesh=mesh,
        compiler_params=pltpu.CompilerParams(use_tc_tiling_on_sc=False),
        out_type=jax.ShapeDtypeStruct((b, s, D), jnp.float32),
        scratch_types=[
            pltpu.VMEM((NBUF, RPC, s), jnp.int32),
            pltpu.VMEM((NBUF, RPC, sp), jnp.int32),
            pltpu.VMEM((NBUF, RPC, s, D), jnp.float32),
            pltpu.SemaphoreType.DMA((NBUF,)),
            pltpu.SemaphoreType.DMA((NBUF,)),
            pltpu.SemaphoreType.DMA((NBUF,)),
        ],
    )
    def k(table_hbm, idx_hbm, out_hbm, raw_b, code_b, rows_b, sem_i, sem_g,
          sem_o):
        wid = lax.axis_index("s") * nc + lax.axis_index("c")
        row0 = wid * rows_w
        zeros16 = jnp.zeros((LANES,), jnp.float32)

        def start_idx(i, sl):
            pltpu.async_copy(
                idx_hbm.at[pl.ds(row0 + i * RPC, RPC)],
                raw_b.at[sl],
                sem_i.at[sl],
            )

        def out_copies(i, sl, start):
            rb = row0 + i * RPC
            for r in range(RPC):
                cp = pltpu.make_async_copy(
                    rows_b.at[sl].at[r],
                    out_hbm.at[rb + r],
                    sem_o.at[sl],
                )
                if start:
                    cp.start()
                else:
                    cp.wait()

        def gathers(i, sl, start):
            code_s = code_b.at[sl]
            rows_s = rows_b.at[sl]
            for r in range(RPC):
                for lo, sz in groups:
                    cp = pltpu.make_async_copy(
                        table_hbm.at[
                            plsc.Indices(
                                code_s.at[r].at[pl.ds(lo, sz)],
                                ignored_value=SENTINEL,
                            )
                        ],
                        rows_s.at[r].at[pl.ds(lo, sz)],
                        sem_g.at[sl],
                    )
                    if start:
                        cp.start()
                    else:
                        cp.wait()

        def prep(i):
            sl = lax.rem(i, NBUF)
            raw_s = raw_b.at[sl]
            code_s = code_b.at[sl]
            rows_s = rows_b.at[sl]

            pltpu.make_async_copy(
                idx_hbm.at[pl.ds(row0 + i * RPC, RPC)], raw_s, sem_i.at[sl]
            ).wait()

            for r in range(RPC):
                for o in voffs:
                    iv = raw_s[r, pl.ds(o, LANES)]
                    code_s[r, pl.ds(o, LANES)] = jnp.where(
                        iv >= VOCAB, iv - VOCAB, SENTINEL
                    )

            @pl.when(i >= NBUF)
            def _():
                out_copies(i - NBUF, sl, start=False)

            def clear_row(r):
                def body(v, c):
                    rows_s[r, v, pl.ds(0, LANES)] = zeros16
                    rows_s[r, v, pl.ds(LANES, LANES)] = zeros16
                    return c

                lax.fori_loop(0, s, body, 0, unroll=10)

            for r in range(RPC):
                clear_row(r)

            gathers(i, sl, start=True)

            @pl.when(i + NBUF < n_chunks)
            def _():
                start_idx(i + NBUF, sl)

        for i in range(NBUF):
            start_idx(i, i)
        prep(0)

        def chunk_body(i, carry):
            @pl.when(i + 1 < n_chunks)
            def _():
                prep(i + 1)

            gathers(i, lax.rem(i, NBUF), start=False)
            out_copies(i, lax.rem(i, NBUF), start=True)
            return carry

        lax.fori_loop(0, n_chunks, chunk_body, 0)

        for j in range(NBUF):
            i = n_chunks - NBUF + j
            out_copies(i, i % NBUF, start=False)

    return k(table, idx2)


def kernel(inputs, embeddings):
    b, s = inputs.shape
    return _lookup(embeddings, inputs.astype(jnp.int32), b, s)
